# baseline (speedup 1.0000x reference)
import functools

import jax
import jax.numpy as jnp
from jax import lax
from jax.experimental import pallas as pl
from jax.experimental.pallas import tpu as pltpu
from jax.experimental.pallas import tpu_sc as plsc

K = 16
N = 100000
NUM_CORES = 2
NUM_SUBCORES = 16
NW = NUM_CORES * NUM_SUBCORES
LANES = 16
CHUNK = 3136
TAIL = N - (NW - 1) * CHUNK
BIGKEY = 1.0e30
NCAND = NW * K


def _sc_topk_body(xs_h, ys_h, zs_h, pc_h, out_h, xv, yv, zv, pv, stg, sem):
    cid = lax.axis_index("c")
    sid = lax.axis_index("s")
    wid = sid * NUM_CORES + cid
    base = wid * CHUNK

    @pl.when(wid < NW - 1)
    def _():
        c1 = pltpu.async_copy(xs_h.at[pl.ds(base, CHUNK)], xv, sem)
        c2 = pltpu.async_copy(ys_h.at[pl.ds(base, CHUNK)], yv, sem)
        c3 = pltpu.async_copy(zs_h.at[pl.ds(base, CHUNK)], zv, sem)
        c1.wait()
        c2.wait()
        c3.wait()

    @pl.when(wid == NW - 1)
    def _():
        c1 = pltpu.async_copy(xs_h.at[pl.ds(base, TAIL)],
                              xv.at[pl.ds(0, TAIL)], sem)
        c2 = pltpu.async_copy(ys_h.at[pl.ds(base, TAIL)],
                              yv.at[pl.ds(0, TAIL)], sem)
        c3 = pltpu.async_copy(zs_h.at[pl.ds(base, TAIL)],
                              zv.at[pl.ds(0, TAIL)], sem)
        c1.wait()
        c2.wait()
        c3.wait()

    pltpu.sync_copy(pc_h, pv)
    px = pv[pl.ds(0, LANES)]
    py = pv[pl.ds(LANES, LANES)]
    pz = pv[pl.ds(2 * LANES, LANES)]

    lane = lax.iota(jnp.int32, LANES)
    gbase = base + lane

    def step(i, carry):
        bk, bv = carry
        off = i * LANES
        dx = xv[pl.ds(off, LANES)] - px
        dy = yv[pl.ds(off, LANES)] - py
        dz = zv[pl.ds(off, LANES)] - pz
        d2 = dx * dx + dy * dy + dz * dz
        d2 = jnp.where(gbase + off < N, d2, BIGKEY)
        nk, nv = plsc.sort_key_val(d2, lane + off)
        rk = lax.rev(nk, (0,))
        rv = lax.rev(nv, (0,))
        take = bk <= rk
        mk = jnp.where(take, bk, rk)
        mv = jnp.where(take, bv, rv)
        sk, sv = plsc.sort_key_val(mk, mv)
        return sk, sv

    bk0 = jnp.full((LANES,), BIGKEY, jnp.float32)
    bv0 = jnp.zeros((LANES,), jnp.int32)
    bk, bv = lax.fori_loop(0, CHUNK // LANES, step, (bk0, bv0))

    fx = plsc.load_gather(xv, [bv])
    fy = plsc.load_gather(yv, [bv])
    fz = plsc.load_gather(zv, [bv])

    stg[pl.ds(0 * LANES, LANES)] = bk
    stg[pl.ds(1 * LANES, LANES)] = (bv + base).astype(jnp.float32)
    stg[pl.ds(2 * LANES, LANES)] = fx
    stg[pl.ds(3 * LANES, LANES)] = fy
    stg[pl.ds(4 * LANES, LANES)] = fz
    pltpu.sync_copy(stg, out_h.at[pl.ds(wid * 5 * LANES, 5 * LANES)])


@functools.cache
def _make_sc_topk():
  return functools.partial(
    pl.kernel,
    out_type=jax.ShapeDtypeStruct((5 * NCAND,), jnp.float32),
    mesh=plsc.VectorSubcoreMesh(core_axis_name="c", subcore_axis_name="s",
                                num_cores=NUM_CORES,
                                num_subcores=NUM_SUBCORES),
    compiler_params=pltpu.CompilerParams(needs_layout_passes=False),
    scratch_types=(
        pltpu.VMEM((CHUNK,), jnp.float32),
        pltpu.VMEM((CHUNK,), jnp.float32),
        pltpu.VMEM((CHUNK,), jnp.float32),
        pltpu.VMEM((3 * LANES,), jnp.float32),
        pltpu.VMEM((5 * LANES,), jnp.float32),
        pltpu.SemaphoreType.DMA,
    ),
  )(_sc_topk_body)


def _tc_finish_body(cand_ref, p_ref, nn_ref, wt_ref, b_ref, out_ref):
    BIG = jnp.float32(3.0e38)
    keys = jnp.sqrt(cand_ref[:, 0 * LANES:1 * LANES])
    gidx = cand_ref[:, 1 * LANES:2 * LANES]
    cx = cand_ref[:, 2 * LANES:3 * LANES]
    cy = cand_ref[:, 3 * LANES:4 * LANES]
    cz = cand_ref[:, 4 * LANES:5 * LANES]

    row_ids = lax.broadcasted_iota(jnp.int32, (K, 1), 0)
    fx = jnp.zeros((K, 1), jnp.float32)
    fy = jnp.zeros((K, 1), jnp.float32)
    fz = jnp.zeros((K, 1), jnp.float32)
    for k in range(K):
        m = jnp.min(keys)
        j = jnp.min(jnp.where(keys == m, gidx, BIG))
        msk = gidx == j
        sel = lambda c: jnp.sum(jnp.where(msk, c, 0.0))
        rk = row_ids == k
        fx = fx + jnp.where(rk, sel(cx), 0.0)
        fy = fy + jnp.where(rk, sel(cy), 0.0)
        fz = fz + jnp.where(rk, sel(cz), 0.0)
        keys = jnp.where(msk, BIG, keys)

    p = p_ref[...]
    nn = nn_ref[...]
    diff = nn - p
    nrm = jnp.sqrt(jnp.sum(diff * diff, axis=1, keepdims=True))
    inp = jnp.concatenate(
        [jnp.broadcast_to(p, (K, 3)), nn, diff, nrm], axis=1)
    wt = wt_ref[...]
    bb = b_ref[...]
    r = bb + jnp.dot(inp, wt, preferred_element_type=jnp.float32)
    out_ref[...] = jnp.concatenate([r, fx, fy, fz], axis=1)


_tc_finish = pl.pallas_call(
    _tc_finish_body,
    out_shape=jax.ShapeDtypeStruct((K, 6), jnp.float32),
)


def kernel(xyz_feat, idx, W, b):
    xs = xyz_feat[:, 0]
    ys = xyz_feat[:, 1]
    zs = xyz_feat[:, 2]
    p = lax.dynamic_slice_in_dim(xyz_feat, idx, 1, axis=0)[0, :3]
    pc = jnp.repeat(p, LANES)

    cand = _make_sc_topk()(xs, ys, zs, pc)

    F = _tc_finish(
        cand.reshape(NW, 5 * LANES),
        p.reshape(1, 3),
        xyz_feat[:K, :3],
        W.T,
        b.reshape(1, 3),
    )
    return F


# speedup vs baseline: 1.0637x; 1.0637x over previous
import functools

import jax
import jax.numpy as jnp
from jax import lax
from jax.experimental import pallas as pl
from jax.experimental.pallas import tpu as pltpu
from jax.experimental.pallas import tpu_sc as plsc

K = 16
N = 100000
NUM_CORES = 2
NUM_SUBCORES = 16
NW = NUM_CORES * NUM_SUBCORES
LANES = 16
CHUNK = 3136
TAIL = N - (NW - 1) * CHUNK
BIGKEY = 1.0e30
NCAND = NW * K


def _sc_topk_body(xs_h, ys_h, zs_h, pc_h, out_h, xv, yv, zv, pv, stg, sem):
    cid = lax.axis_index("c")
    sid = lax.axis_index("s")
    wid = sid * NUM_CORES + cid
    base = wid * CHUNK

    @pl.when(wid < NW - 1)
    def _():
        c1 = pltpu.async_copy(xs_h.at[pl.ds(base, CHUNK)], xv, sem)
        c2 = pltpu.async_copy(ys_h.at[pl.ds(base, CHUNK)], yv, sem)
        c3 = pltpu.async_copy(zs_h.at[pl.ds(base, CHUNK)], zv, sem)
        c1.wait()
        c2.wait()
        c3.wait()

    @pl.when(wid == NW - 1)
    def _():
        c1 = pltpu.async_copy(xs_h.at[pl.ds(base, TAIL)],
                              xv.at[pl.ds(0, TAIL)], sem)
        c2 = pltpu.async_copy(ys_h.at[pl.ds(base, TAIL)],
                              yv.at[pl.ds(0, TAIL)], sem)
        c3 = pltpu.async_copy(zs_h.at[pl.ds(base, TAIL)],
                              zv.at[pl.ds(0, TAIL)], sem)
        c1.wait()
        c2.wait()
        c3.wait()

    pltpu.sync_copy(pc_h, pv)
    px = pv[pl.ds(0, LANES)]
    py = pv[pl.ds(LANES, LANES)]
    pz = pv[pl.ds(2 * LANES, LANES)]

    lane = lax.iota(jnp.int32, LANES)
    gbase = base + lane

    def step(i, carry):
        bk, bv = carry
        off = i * LANES
        dx = xv[pl.ds(off, LANES)] - px
        dy = yv[pl.ds(off, LANES)] - py
        dz = zv[pl.ds(off, LANES)] - pz
        d2 = dx * dx + dy * dy + dz * dz
        d2 = jnp.where(gbase + off < N, d2, BIGKEY)
        nk, nv = plsc.sort_key_val(d2, lane + off)
        rk = lax.rev(nk, (0,))
        rv = lax.rev(nv, (0,))
        take = bk <= rk
        mk = jnp.where(take, bk, rk)
        mv = jnp.where(take, bv, rv)
        sk, sv = plsc.sort_key_val(mk, mv)
        return sk, sv

    bk0 = jnp.full((LANES,), BIGKEY, jnp.float32)
    bv0 = jnp.zeros((LANES,), jnp.int32)
    bk, bv = lax.fori_loop(0, CHUNK // LANES, step, (bk0, bv0))

    fx = plsc.load_gather(xv, [bv])
    fy = plsc.load_gather(yv, [bv])
    fz = plsc.load_gather(zv, [bv])

    woff = wid * LANES
    stg[...] = bk
    pltpu.sync_copy(stg, out_h.at[pl.ds(0 * NCAND + woff, LANES)])
    stg[...] = (bv + base).astype(jnp.float32)
    pltpu.sync_copy(stg, out_h.at[pl.ds(1 * NCAND + woff, LANES)])
    stg[...] = fx
    pltpu.sync_copy(stg, out_h.at[pl.ds(2 * NCAND + woff, LANES)])
    stg[...] = fy
    pltpu.sync_copy(stg, out_h.at[pl.ds(3 * NCAND + woff, LANES)])
    stg[...] = fz
    pltpu.sync_copy(stg, out_h.at[pl.ds(4 * NCAND + woff, LANES)])


@functools.cache
def _make_sc_topk():
  return functools.partial(
    pl.kernel,
    out_type=jax.ShapeDtypeStruct((5 * NCAND,), jnp.float32),
    mesh=plsc.VectorSubcoreMesh(core_axis_name="c", subcore_axis_name="s",
                                num_cores=NUM_CORES,
                                num_subcores=NUM_SUBCORES),
    compiler_params=pltpu.CompilerParams(needs_layout_passes=False),
    scratch_types=(
        pltpu.VMEM((CHUNK,), jnp.float32),
        pltpu.VMEM((CHUNK,), jnp.float32),
        pltpu.VMEM((CHUNK,), jnp.float32),
        pltpu.VMEM((3 * LANES,), jnp.float32),
        pltpu.VMEM((LANES,), jnp.float32),
        pltpu.SemaphoreType.DMA,
    ),
  )(_sc_topk_body)


def _tc_finish_body(cand_ref, p_ref, nn_ref, wt_ref, b_ref, out_ref):
    BIG = jnp.float32(3.0e38)
    keys = jnp.sqrt(cand_ref[0:4, :])
    gidx = cand_ref[4:8, :]
    cx = cand_ref[8:12, :]
    cy = cand_ref[12:16, :]
    cz = cand_ref[16:20, :]

    row_ids = lax.broadcasted_iota(jnp.int32, (K, 1), 0)
    fx = jnp.zeros((K, 1), jnp.float32)
    fy = jnp.zeros((K, 1), jnp.float32)
    fz = jnp.zeros((K, 1), jnp.float32)
    for k in range(K):
        m = jnp.min(keys)
        j = jnp.min(jnp.where(keys == m, gidx, BIG))
        msk = gidx == j
        sel = lambda c: jnp.sum(jnp.where(msk, c, 0.0))
        rk = row_ids == k
        fx = fx + jnp.where(rk, sel(cx), 0.0)
        fy = fy + jnp.where(rk, sel(cy), 0.0)
        fz = fz + jnp.where(rk, sel(cz), 0.0)
        keys = jnp.where(msk, BIG, keys)

    p = p_ref[...]
    nn = nn_ref[...]
    diff = nn - p
    nrm = jnp.sqrt(jnp.sum(diff * diff, axis=1, keepdims=True))
    inp = jnp.concatenate(
        [jnp.broadcast_to(p, (K, 3)), nn, diff, nrm], axis=1)
    wt = wt_ref[...]
    bb = b_ref[...]
    r = bb + jnp.dot(inp, wt, preferred_element_type=jnp.float32)
    out_ref[...] = jnp.concatenate([r, fx, fy, fz], axis=1)


_tc_finish = pl.pallas_call(
    _tc_finish_body,
    out_shape=jax.ShapeDtypeStruct((K, 6), jnp.float32),
)


def kernel(xyz_feat, idx, W, b):
    xs = xyz_feat[:, 0]
    ys = xyz_feat[:, 1]
    zs = xyz_feat[:, 2]
    p = lax.dynamic_slice_in_dim(xyz_feat, idx, 1, axis=0)[0, :3]
    pc = jnp.repeat(p, LANES)

    cand = _make_sc_topk()(xs, ys, zs, pc)

    F = _tc_finish(
        cand.reshape(20, 128),
        p.reshape(1, 3),
        xyz_feat[:K, :3],
        W.T,
        b.reshape(1, 3),
    )
    return F
